# Initial kernel scaffold; baseline (speedup 1.0000x reference)
import jax
import jax.numpy as jnp
from jax.experimental import pallas as pl


def kernel(x_scalar, x_vector, edge_index, rbf, rsh, envelop_para, W_rbf1, W_rbf2, W_s1, W_s2, W_q, b_q, W_k, b_k, W_v1, b_v1, W_v2, b_v2):
    raise NotImplementedError("write your pallas kernel here")


# speedup vs baseline: 6.0297x; 6.0297x over previous
import math

import jax
import jax.numpy as jnp
import numpy as np
from jax.experimental import pallas as pl

N_NODES = 10000
N_EDGES = 160000
C = 256
HEADS = 8
MATCH = 32
OUT = 32
RBF_DIM = 20
IRR_DIM = 176
NUM_IR = 48

_F32 = jnp.float32


def _su2_gen_np(l):
    m = np.arange(-l, l + 1)
    d = 2 * l + 1
    Jp = np.zeros((d, d), dtype=complex)
    for i in range(d - 1):
        mm = m[i]
        Jp[i + 1, i] = np.sqrt(l * (l + 1) - mm * (mm + 1))
    Jm = Jp.conj().T
    Jx = (Jp + Jm) / 2.0
    Jy = (Jp - Jm) / 2j
    Jz = np.diag(m).astype(complex)
    return [Jx, Jy, Jz]


def _c2r_np(l):
    d = 2 * l + 1
    U = np.zeros((d, d), dtype=complex)
    for m in range(-l, l + 1):
        i = m + l
        if m > 0:
            U[i, l + m] = (-1) ** m / np.sqrt(2)
            U[i, l - m] = 1.0 / np.sqrt(2)
        elif m == 0:
            U[i, l] = 1.0
        else:
            U[i, l + m] = 1j / np.sqrt(2)
            U[i, l - m] = -1j * (-1) ** m / np.sqrt(2)
    return U


def _real_gens_np(l):
    U = _c2r_np(l)
    return [np.real(U @ (-1j * J) @ U.conj().T) for J in _su2_gen_np(l)]


def _w3j_np(l):
    d = 2 * l + 1
    I = np.eye(d)
    rows = []
    for X in _real_gens_np(l):
        M = (np.kron(np.kron(X, I), I) + np.kron(np.kron(I, X), I)
             + np.kron(np.kron(I, I), X))
        rows.append(M)
    M = np.concatenate(rows, axis=0)
    _, _, vt = np.linalg.svd(M)
    Cc = vt[-1].reshape(d, d, d)
    return Cc / np.linalg.norm(Cc)


_W3J1 = _w3j_np(1).astype(np.float32)
_W3J2 = _w3j_np(2).astype(np.float32)
_C1 = 3.0 / math.sqrt(32.0)
_C2 = 5.0 / math.sqrt(16.0)

_T1 = [(i, j, k, float(_W3J1[i, j, k]))
       for i in range(3) for j in range(3) for k in range(3)
       if abs(_W3J1[i, j, k]) > 1e-7]
_T2 = [(i, j, k, float(_W3J2[i, j, k]))
       for i in range(5) for j in range(5) for k in range(5)
       if abs(_W3J2[i, j, k]) > 1e-7]

_EXP48_np = np.zeros((NUM_IR, IRR_DIM), np.float32)
for _u in range(32):
    _EXP48_np[_u, 3 * _u:3 * _u + 3] = 1.0
for _u in range(16):
    _EXP48_np[32 + _u, 96 + 5 * _u:96 + 5 * _u + 5] = 1.0

_EXP8_np = np.zeros((HEADS, C), np.float32)
for _h in range(HEADS):
    _EXP8_np[_h, 32 * _h:32 * _h + 32] = 1.0

_SUM1_np = np.zeros((96, 8), np.float32)
for _v in range(32):
    for _i in range(3):
        _SUM1_np[3 * _v + _i, _i] = 1.0
_SUM2_np = np.zeros((80, 8), np.float32)
for _v in range(16):
    for _i in range(5):
        _SUM2_np[5 * _v + _i, _i] = 1.0

_S1all_np = np.zeros((96, 96), np.float32)
for _j in range(3):
    for _v in range(32):
        _S1all_np[3 * _v + _j, 32 * _j + _v] = 1.0
_SCAT1_np = _S1all_np.T.copy()

_S2all_np = np.zeros((80, 80), np.float32)
for _j in range(5):
    for _v in range(16):
        _S2all_np[5 * _v + _j, 16 * _j + _v] = 1.0
_SCAT2_np = _S2all_np.T.copy()

_EXP48 = jnp.asarray(_EXP48_np)
_EXP8 = jnp.asarray(_EXP8_np)
_SUM1 = jnp.asarray(_SUM1_np)
_SUM2 = jnp.asarray(_SUM2_np)
_S1all = jnp.asarray(_S1all_np)
_SCAT1 = jnp.asarray(_SCAT1_np)
_S2all = jnp.asarray(_S2all_np)
_SCAT2 = jnp.asarray(_SCAT2_np)


_BN = 1000


def _dense_body(x_ref, rbf_ref, rsh_ref, env_ref,
                wr1, wr2, ws1, ws2, wq, bq, wk, bk, wv1, bv1, wv2, bv2,
                exp48,
                q_ref, k_ref, v_ref, mixed_ref):
    x = x_ref[...]
    t = jnp.dot(rbf_ref[...], wr1[...], preferred_element_type=_F32)
    xsrc = x * env_ref[...] * jnp.dot(t, wr2[...], preferred_element_type=_F32)
    q_ref[...] = jnp.dot(xsrc, wq[...], preferred_element_type=_F32) + bq[...]
    k_ref[...] = jnp.dot(x, wk[...], preferred_element_type=_F32) + bk[...]
    h = jnp.dot(xsrc, wv1[...], preferred_element_type=_F32) + bv1[...]
    h = h * jax.nn.sigmoid(h)
    v_ref[...] = jnp.dot(h, wv2[...], preferred_element_type=_F32) + bv2[...]
    scal = jnp.dot(jnp.dot(x, ws1[...], preferred_element_type=_F32),
                   ws2[...], preferred_element_type=_F32)
    mixed_ref[...] = rsh_ref[...] * jnp.dot(scal, exp48[...],
                                            preferred_element_type=_F32)


def _blk(shape):
    return pl.BlockSpec(shape, lambda i: (i, 0))


def _full(shape):
    return pl.BlockSpec(shape, lambda i: (0, 0))


def _dense_stage(x_scalar, rbf, rsh, envelop_para,
                 W_rbf1, W_rbf2, W_s1, W_s2, W_q, b_q, W_k, b_k,
                 W_v1, b_v1, W_v2, b_v2):
    grid = (N_NODES // _BN,)
    in_specs = [
        _blk((_BN, C)), _blk((_BN, RBF_DIM)), _blk((_BN, IRR_DIM)),
        _blk((_BN, 1)),
        _full((RBF_DIM, C)), _full((C, C)), _full((C, C)),
        _full((C, NUM_IR)), _full((C, C)), _full((1, C)),
        _full((C, C)), _full((1, C)), _full((C, C)), _full((1, C)),
        _full((C, C)), _full((1, C)), _full((NUM_IR, IRR_DIM)),
    ]
    out_specs = [_blk((_BN, C)), _blk((_BN, C)), _blk((_BN, C)),
                 _blk((_BN, IRR_DIM))]
    out_shape = [jax.ShapeDtypeStruct((N_NODES, C), _F32)] * 3 + [
        jax.ShapeDtypeStruct((N_NODES, IRR_DIM), _F32)]
    return pl.pallas_call(
        _dense_body, grid=grid, in_specs=in_specs, out_specs=out_specs,
        out_shape=out_shape,
    )(x_scalar, rbf, rsh, envelop_para,
      W_rbf1, W_rbf2, W_s1, W_s2, W_q, b_q.reshape(1, C), W_k,
      b_k.reshape(1, C), W_v1, b_v1.reshape(1, C), W_v2, b_v2.reshape(1, C),
      _EXP48)


def _final_body(sacc_ref, den_ref, agg_ref, x_ref, xv_ref,
                exp8, sum1, sum2, s1all, scat1, s2all, scat2,
                so_ref, vo_ref):
    sacc = sacc_ref[...]
    den = jnp.dot(den_ref[...], exp8[...], preferred_element_type=_F32)
    so_ref[...] = sacc / (den + 1e-16) + x_ref[...]

    agg = agg_ref[...]
    xv = xv_ref[...]
    agg1 = agg[:, :96]
    agg2 = agg[:, 96:176]
    b1 = xv[:, :96]
    b2 = xv[:, 96:176]
    A1 = jnp.dot(agg1, sum1[...], preferred_element_type=_F32)
    A2 = jnp.dot(agg2, sum2[...], preferred_element_type=_F32)
    B1 = jnp.dot(b1, s1all[...], preferred_element_type=_F32)
    B2 = jnp.dot(b2, s2all[...], preferred_element_type=_F32)

    cols1 = [jnp.zeros((sacc.shape[0], 32), _F32) for _ in range(3)]
    for (i, j, k, w) in _T1:
        cols1[k] = cols1[k] + (_C1 * w) * A1[:, i:i + 1] * B1[:, 32 * j:32 * j + 32]
    o1 = jnp.dot(jnp.concatenate(cols1, axis=1), scat1[...],
                 preferred_element_type=_F32)

    cols2 = [jnp.zeros((sacc.shape[0], 16), _F32) for _ in range(5)]
    for (i, j, k, w) in _T2:
        cols2[k] = cols2[k] + (_C2 * w) * A2[:, i:i + 1] * B2[:, 16 * j:16 * j + 16]
    o2 = jnp.dot(jnp.concatenate(cols2, axis=1), scat2[...],
                 preferred_element_type=_F32)

    vo_ref[:, :96] = o1 + xv[:, :96]
    vo_ref[:, 96:176] = o2 + xv[:, 96:176]


def _final_stage(sacc, den, agg, x_scalar, x_vector):
    grid = (N_NODES // _BN,)
    in_specs = [
        _blk((_BN, C)), _blk((_BN, HEADS)), _blk((_BN, IRR_DIM)),
        _blk((_BN, C)), _blk((_BN, IRR_DIM)),
        _full((HEADS, C)), _full((96, 8)), _full((80, 8)),
        _full((96, 96)), _full((96, 96)), _full((80, 80)), _full((80, 80)),
    ]
    out_specs = [_blk((_BN, C)), _blk((_BN, IRR_DIM))]
    out_shape = [jax.ShapeDtypeStruct((N_NODES, C), _F32),
                 jax.ShapeDtypeStruct((N_NODES, IRR_DIM), _F32)]
    return pl.pallas_call(
        _final_body, grid=grid, in_specs=in_specs, out_specs=out_specs,
        out_shape=out_shape,
    )(sacc, den, agg, x_scalar, x_vector,
      _EXP8, _SUM1, _SUM2, _S1all, _SCAT1, _S2all, _SCAT2)


def kernel(x_scalar, x_vector, edge_index, rbf, rsh, envelop_para,
           W_rbf1, W_rbf2, W_s1, W_s2, W_q, b_q, W_k, b_k,
           W_v1, b_v1, W_v2, b_v2):
    src = edge_index[0]
    dst = edge_index[1]

    q, k, v, mixed = _dense_stage(
        x_scalar, rbf, rsh, envelop_para,
        W_rbf1, W_rbf2, W_s1, W_s2, W_q, b_q, W_k, b_k,
        W_v1, b_v1, W_v2, b_v2)

    qd = q[dst]
    ks = k[src]
    att = (qd * ks).reshape(N_EDGES, HEADS, MATCH).sum(-1) / math.sqrt(MATCH)
    mx = jax.ops.segment_max(att, dst, num_segments=N_NODES)
    mx = jnp.where(jnp.isfinite(mx), mx, 0.0)
    e = jnp.exp(att - mx[dst])
    den = jax.ops.segment_sum(e, dst, num_segments=N_NODES)
    sacc = jax.ops.segment_sum(
        (e[:, :, None] * v[src].reshape(N_EDGES, HEADS, OUT)).reshape(
            N_EDGES, HEADS * OUT),
        dst, num_segments=N_NODES)
    agg = jax.ops.segment_sum(mixed[src], dst, num_segments=N_NODES)

    return _final_stage(sacc, den, agg, x_scalar, x_vector)


# speedup vs baseline: 6.3347x; 1.0506x over previous
import functools
import math

import jax
import jax.numpy as jnp
import numpy as np
from jax import lax
from jax.experimental import pallas as pl
from jax.experimental.pallas import tpu as pltpu
from jax.experimental.pallas import tpu_sc as plsc

N_NODES = 10000
N_EDGES = 160000
C = 256
HEADS = 8
MATCH = 32
OUT = 32
RBF_DIM = 20
IRR_DIM = 176
NUM_IR = 48

_F32 = jnp.float32


def _su2_gen_np(l):
    m = np.arange(-l, l + 1)
    d = 2 * l + 1
    Jp = np.zeros((d, d), dtype=complex)
    for i in range(d - 1):
        mm = m[i]
        Jp[i + 1, i] = np.sqrt(l * (l + 1) - mm * (mm + 1))
    Jm = Jp.conj().T
    Jx = (Jp + Jm) / 2.0
    Jy = (Jp - Jm) / 2j
    Jz = np.diag(m).astype(complex)
    return [Jx, Jy, Jz]


def _c2r_np(l):
    d = 2 * l + 1
    U = np.zeros((d, d), dtype=complex)
    for m in range(-l, l + 1):
        i = m + l
        if m > 0:
            U[i, l + m] = (-1) ** m / np.sqrt(2)
            U[i, l - m] = 1.0 / np.sqrt(2)
        elif m == 0:
            U[i, l] = 1.0
        else:
            U[i, l + m] = 1j / np.sqrt(2)
            U[i, l - m] = -1j * (-1) ** m / np.sqrt(2)
    return U


def _real_gens_np(l):
    U = _c2r_np(l)
    return [np.real(U @ (-1j * J) @ U.conj().T) for J in _su2_gen_np(l)]


def _w3j_np(l):
    d = 2 * l + 1
    I = np.eye(d)
    rows = []
    for X in _real_gens_np(l):
        M = (np.kron(np.kron(X, I), I) + np.kron(np.kron(I, X), I)
             + np.kron(np.kron(I, I), X))
        rows.append(M)
    M = np.concatenate(rows, axis=0)
    _, _, vt = np.linalg.svd(M)
    Cc = vt[-1].reshape(d, d, d)
    return Cc / np.linalg.norm(Cc)


_W3J1 = _w3j_np(1).astype(np.float32)
_W3J2 = _w3j_np(2).astype(np.float32)
_C1 = 3.0 / math.sqrt(32.0)
_C2 = 5.0 / math.sqrt(16.0)

_T1 = [(i, j, k, float(_W3J1[i, j, k]))
       for i in range(3) for j in range(3) for k in range(3)
       if abs(_W3J1[i, j, k]) > 1e-7]
_T2 = [(i, j, k, float(_W3J2[i, j, k]))
       for i in range(5) for j in range(5) for k in range(5)
       if abs(_W3J2[i, j, k]) > 1e-7]

_EXP48_np = np.zeros((NUM_IR, IRR_DIM), np.float32)
for _u in range(32):
    _EXP48_np[_u, 3 * _u:3 * _u + 3] = 1.0
for _u in range(16):
    _EXP48_np[32 + _u, 96 + 5 * _u:96 + 5 * _u + 5] = 1.0

_EXP8_np = np.zeros((HEADS, C), np.float32)
for _h in range(HEADS):
    _EXP8_np[_h, 32 * _h:32 * _h + 32] = 1.0

_SUM1_np = np.zeros((96, 8), np.float32)
for _v in range(32):
    for _i in range(3):
        _SUM1_np[3 * _v + _i, _i] = 1.0
_SUM2_np = np.zeros((80, 8), np.float32)
for _v in range(16):
    for _i in range(5):
        _SUM2_np[5 * _v + _i, _i] = 1.0

_S1all_np = np.zeros((96, 96), np.float32)
for _j in range(3):
    for _v in range(32):
        _S1all_np[3 * _v + _j, 32 * _j + _v] = 1.0
_SCAT1_np = _S1all_np.T.copy()

_S2all_np = np.zeros((80, 80), np.float32)
for _j in range(5):
    for _v in range(16):
        _S2all_np[5 * _v + _j, 16 * _j + _v] = 1.0
_SCAT2_np = _S2all_np.T.copy()

_EXP48 = jnp.asarray(_EXP48_np)
_EXP8 = jnp.asarray(_EXP8_np)
_SUM1 = jnp.asarray(_SUM1_np)
_SUM2 = jnp.asarray(_SUM2_np)
_S1all = jnp.asarray(_S1all_np)
_SCAT1 = jnp.asarray(_SCAT1_np)
_S2all = jnp.asarray(_S2all_np)
_SCAT2 = jnp.asarray(_SCAT2_np)


_BN = 1000


def _dense_body(x_ref, rbf_ref, rsh_ref, env_ref,
                wr1, wr2, ws1, ws2, wq, bq, wk, bk, wv1, bv1, wv2, bv2,
                exp48,
                q_ref, k_ref, v_ref, m0_ref, m1_ref):
    x = x_ref[...]
    t = jnp.dot(rbf_ref[...], wr1[...], preferred_element_type=_F32)
    xsrc = x * env_ref[...] * jnp.dot(t, wr2[...], preferred_element_type=_F32)
    q_ref[...] = jnp.dot(xsrc, wq[...], preferred_element_type=_F32) + bq[...]
    k_ref[...] = jnp.dot(x, wk[...], preferred_element_type=_F32) + bk[...]
    h = jnp.dot(xsrc, wv1[...], preferred_element_type=_F32) + bv1[...]
    h = h * jax.nn.sigmoid(h)
    v_ref[...] = jnp.dot(h, wv2[...], preferred_element_type=_F32) + bv2[...]
    scal = jnp.dot(jnp.dot(x, ws1[...], preferred_element_type=_F32),
                   ws2[...], preferred_element_type=_F32)
    mixed = rsh_ref[...] * jnp.dot(scal, exp48[...],
                                   preferred_element_type=_F32)
    m0_ref[:, :96] = mixed[:, :96]
    m0_ref[:, 96:128] = jnp.zeros_like(mixed[:, :32])
    m1_ref[:, :80] = mixed[:, 96:176]
    m1_ref[:, 80:128] = jnp.zeros_like(mixed[:, :48])


def _blk(shape):
    return pl.BlockSpec(shape, lambda i: (i, 0))


def _full(shape):
    return pl.BlockSpec(shape, lambda i: (0, 0))


def _dense_stage(x_scalar, rbf, rsh, envelop_para,
                 W_rbf1, W_rbf2, W_s1, W_s2, W_q, b_q, W_k, b_k,
                 W_v1, b_v1, W_v2, b_v2):
    grid = (N_NODES // _BN,)
    in_specs = [
        _blk((_BN, C)), _blk((_BN, RBF_DIM)), _blk((_BN, IRR_DIM)),
        _blk((_BN, 1)),
        _full((RBF_DIM, C)), _full((C, C)), _full((C, C)),
        _full((C, NUM_IR)), _full((C, C)), _full((1, C)),
        _full((C, C)), _full((1, C)), _full((C, C)), _full((1, C)),
        _full((C, C)), _full((1, C)), _full((NUM_IR, IRR_DIM)),
    ]
    out_specs = [_blk((_BN, C)), _blk((_BN, C)), _blk((_BN, C)),
                 _blk((_BN, 128)), _blk((_BN, 128))]
    out_shape = [jax.ShapeDtypeStruct((N_NODES, C), _F32)] * 3 + [
        jax.ShapeDtypeStruct((N_NODES, 128), _F32)] * 2
    return pl.pallas_call(
        _dense_body, grid=grid, in_specs=in_specs, out_specs=out_specs,
        out_shape=out_shape,
    )(x_scalar, rbf, rsh, envelop_para,
      W_rbf1, W_rbf2, W_s1, W_s2, W_q, b_q.reshape(1, C), W_k,
      b_k.reshape(1, C), W_v1, b_v1.reshape(1, C), W_v2, b_v2.reshape(1, C),
      _EXP48)


_CH = 128
_NCH = -(-N_EDGES // (16 * _CH))
_EPW = _NCH * _CH
_EP = 16 * _EPW
_RPW = 632
_NPAD = 16 * _RPW
_DUMMY = N_NODES

_SC_MESH = plsc.VectorSubcoreMesh(core_axis_name="c", subcore_axis_name="s")


def _agg_body(m2_hbm, src3, dst3, zrows, out_hbm, sidx, didx, rows, acc, sem):
    c = lax.axis_index("c")
    s = lax.axis_index("s")
    pltpu.sync_copy(zrows, acc.at[pl.ds(s * _RPW, _RPW)])
    plsc.subcore_barrier()

    def chunk(j, carry):
        pltpu.sync_copy(src3.at[s, j], sidx)
        pltpu.sync_copy(dst3.at[s, j], didx)
        pltpu.async_copy(m2_hbm.at[c].at[sidx], rows, sem).wait()
        pltpu.sync_copy(rows, acc.at[didx], add=True)
        return carry

    lax.fori_loop(0, _NCH, chunk, 0)
    plsc.subcore_barrier()
    pltpu.sync_copy(acc.at[pl.ds(s * _RPW, _RPW)],
                    out_hbm.at[c, pl.ds(s * _RPW, _RPW)])


def _agg_stage(m2, src3, dst3, zrows):
    return pl.kernel(
        _agg_body,
        out_type=jax.ShapeDtypeStruct((2, _NPAD, 128), _F32),
        mesh=_SC_MESH,
        scratch_types=[
            pltpu.VMEM((_CH,), jnp.int32),
            pltpu.VMEM((_CH,), jnp.int32),
            pltpu.VMEM((_CH, 128), _F32),
            pltpu.VMEM_SHARED((_NPAD, 128), _F32),
            pltpu.SemaphoreType.DMA,
        ],
    )(m2, src3, dst3, zrows)


def _final_body(sacc_ref, den_ref, agg_ref, x_ref, xv_ref,
                exp8, sum1, sum2, s1all, scat1, s2all, scat2,
                so_ref, vo_ref):
    sacc = sacc_ref[...]
    den = jnp.dot(den_ref[...], exp8[...], preferred_element_type=_F32)
    so_ref[...] = sacc / (den + 1e-16) + x_ref[...]

    agg = agg_ref[...]
    xv = xv_ref[...]
    agg1 = agg[:, :96]
    agg2 = agg[:, 96:176]
    b1 = xv[:, :96]
    b2 = xv[:, 96:176]
    A1 = jnp.dot(agg1, sum1[...], preferred_element_type=_F32)
    A2 = jnp.dot(agg2, sum2[...], preferred_element_type=_F32)
    B1 = jnp.dot(b1, s1all[...], preferred_element_type=_F32)
    B2 = jnp.dot(b2, s2all[...], preferred_element_type=_F32)

    cols1 = [jnp.zeros((sacc.shape[0], 32), _F32) for _ in range(3)]
    for (i, j, k, w) in _T1:
        cols1[k] = cols1[k] + (_C1 * w) * A1[:, i:i + 1] * B1[:, 32 * j:32 * j + 32]
    o1 = jnp.dot(jnp.concatenate(cols1, axis=1), scat1[...],
                 preferred_element_type=_F32)

    cols2 = [jnp.zeros((sacc.shape[0], 16), _F32) for _ in range(5)]
    for (i, j, k, w) in _T2:
        cols2[k] = cols2[k] + (_C2 * w) * A2[:, i:i + 1] * B2[:, 16 * j:16 * j + 16]
    o2 = jnp.dot(jnp.concatenate(cols2, axis=1), scat2[...],
                 preferred_element_type=_F32)

    vo_ref[:, :96] = o1 + xv[:, :96]
    vo_ref[:, 96:176] = o2 + xv[:, 96:176]


def _final_stage(sacc, den, agg, x_scalar, x_vector):
    grid = (N_NODES // _BN,)
    in_specs = [
        _blk((_BN, C)), _blk((_BN, HEADS)), _blk((_BN, IRR_DIM)),
        _blk((_BN, C)), _blk((_BN, IRR_DIM)),
        _full((HEADS, C)), _full((96, 8)), _full((80, 8)),
        _full((96, 96)), _full((96, 96)), _full((80, 80)), _full((80, 80)),
    ]
    out_specs = [_blk((_BN, C)), _blk((_BN, IRR_DIM))]
    out_shape = [jax.ShapeDtypeStruct((N_NODES, C), _F32),
                 jax.ShapeDtypeStruct((N_NODES, IRR_DIM), _F32)]
    return pl.pallas_call(
        _final_body, grid=grid, in_specs=in_specs, out_specs=out_specs,
        out_shape=out_shape,
    )(sacc, den, agg, x_scalar, x_vector,
      _EXP8, _SUM1, _SUM2, _S1all, _SCAT1, _S2all, _SCAT2)


def kernel(x_scalar, x_vector, edge_index, rbf, rsh, envelop_para,
           W_rbf1, W_rbf2, W_s1, W_s2, W_q, b_q, W_k, b_k,
           W_v1, b_v1, W_v2, b_v2):
    src = edge_index[0]
    dst = edge_index[1]

    q, k, v, m0, m1 = _dense_stage(
        x_scalar, rbf, rsh, envelop_para,
        W_rbf1, W_rbf2, W_s1, W_s2, W_q, b_q, W_k, b_k,
        W_v1, b_v1, W_v2, b_v2)

    qd = q[dst]
    ks = k[src]
    att = (qd * ks).reshape(N_EDGES, HEADS, MATCH).sum(-1) / math.sqrt(MATCH)
    mx = jax.ops.segment_max(att, dst, num_segments=N_NODES)
    mx = jnp.where(jnp.isfinite(mx), mx, 0.0)
    e = jnp.exp(att - mx[dst])
    den = jax.ops.segment_sum(e, dst, num_segments=N_NODES)
    sacc = jax.ops.segment_sum(
        (e[:, :, None] * v[src].reshape(N_EDGES, HEADS, OUT)).reshape(
            N_EDGES, HEADS * OUT),
        dst, num_segments=N_NODES)
    pad = _EP - N_EDGES
    src3 = jnp.concatenate(
        [src, jnp.zeros((pad,), jnp.int32)]).reshape(16, _NCH, _CH)
    dst3 = jnp.concatenate(
        [dst, jnp.full((pad,), _DUMMY, jnp.int32)]).reshape(16, _NCH, _CH)
    zrows = jnp.zeros((_RPW, 128), _F32)
    m2 = jnp.stack([m0, m1])
    agg_out = _agg_stage(m2, src3, dst3, zrows)
    agg = jnp.concatenate(
        [agg_out[0, :N_NODES, :96], agg_out[1, :N_NODES, :80]], axis=1)

    return _final_stage(sacc, den, agg, x_scalar, x_vector)


# speedup vs baseline: 12.4887x; 1.9715x over previous
import functools
import math

import jax
import jax.numpy as jnp
import numpy as np
from jax import lax
from jax.experimental import pallas as pl
from jax.experimental.pallas import tpu as pltpu
from jax.experimental.pallas import tpu_sc as plsc

N_NODES = 10000
N_EDGES = 160000
C = 256
HEADS = 8
MATCH = 32
OUT = 32
RBF_DIM = 20
IRR_DIM = 176
NUM_IR = 48

_F32 = jnp.float32


def _su2_gen_np(l):
    m = np.arange(-l, l + 1)
    d = 2 * l + 1
    Jp = np.zeros((d, d), dtype=complex)
    for i in range(d - 1):
        mm = m[i]
        Jp[i + 1, i] = np.sqrt(l * (l + 1) - mm * (mm + 1))
    Jm = Jp.conj().T
    Jx = (Jp + Jm) / 2.0
    Jy = (Jp - Jm) / 2j
    Jz = np.diag(m).astype(complex)
    return [Jx, Jy, Jz]


def _c2r_np(l):
    d = 2 * l + 1
    U = np.zeros((d, d), dtype=complex)
    for m in range(-l, l + 1):
        i = m + l
        if m > 0:
            U[i, l + m] = (-1) ** m / np.sqrt(2)
            U[i, l - m] = 1.0 / np.sqrt(2)
        elif m == 0:
            U[i, l] = 1.0
        else:
            U[i, l + m] = 1j / np.sqrt(2)
            U[i, l - m] = -1j * (-1) ** m / np.sqrt(2)
    return U


def _real_gens_np(l):
    U = _c2r_np(l)
    return [np.real(U @ (-1j * J) @ U.conj().T) for J in _su2_gen_np(l)]


def _w3j_np(l):
    d = 2 * l + 1
    I = np.eye(d)
    rows = []
    for X in _real_gens_np(l):
        M = (np.kron(np.kron(X, I), I) + np.kron(np.kron(I, X), I)
             + np.kron(np.kron(I, I), X))
        rows.append(M)
    M = np.concatenate(rows, axis=0)
    _, _, vt = np.linalg.svd(M)
    Cc = vt[-1].reshape(d, d, d)
    return Cc / np.linalg.norm(Cc)


_W3J1 = _w3j_np(1).astype(np.float32)
_W3J2 = _w3j_np(2).astype(np.float32)
_C1 = 3.0 / math.sqrt(32.0)
_C2 = 5.0 / math.sqrt(16.0)

_T1 = [(i, j, k, float(_W3J1[i, j, k]))
       for i in range(3) for j in range(3) for k in range(3)
       if abs(_W3J1[i, j, k]) > 1e-7]
_T2 = [(i, j, k, float(_W3J2[i, j, k]))
       for i in range(5) for j in range(5) for k in range(5)
       if abs(_W3J2[i, j, k]) > 1e-7]

_EXP48_np = np.zeros((NUM_IR, IRR_DIM), np.float32)
for _u in range(32):
    _EXP48_np[_u, 3 * _u:3 * _u + 3] = 1.0
for _u in range(16):
    _EXP48_np[32 + _u, 96 + 5 * _u:96 + 5 * _u + 5] = 1.0

_EXP8_np = np.zeros((HEADS, C), np.float32)
for _h in range(HEADS):
    _EXP8_np[_h, 32 * _h:32 * _h + 32] = 1.0

_SUM1_np = np.zeros((96, 8), np.float32)
for _v in range(32):
    for _i in range(3):
        _SUM1_np[3 * _v + _i, _i] = 1.0
_SUM2_np = np.zeros((80, 8), np.float32)
for _v in range(16):
    for _i in range(5):
        _SUM2_np[5 * _v + _i, _i] = 1.0

_S1all_np = np.zeros((96, 96), np.float32)
for _j in range(3):
    for _v in range(32):
        _S1all_np[3 * _v + _j, 32 * _j + _v] = 1.0
_SCAT1_np = _S1all_np.T.copy()

_S2all_np = np.zeros((80, 80), np.float32)
for _j in range(5):
    for _v in range(16):
        _S2all_np[5 * _v + _j, 16 * _j + _v] = 1.0
_SCAT2_np = _S2all_np.T.copy()

_EXP48 = jnp.asarray(_EXP48_np)
_EXP8 = jnp.asarray(_EXP8_np)
_EXP8T = jnp.asarray(_EXP8_np.T.copy())
_EXPA_np = np.zeros((HEADS, 128), np.float32)
_EXPB_np = np.zeros((HEADS, 128), np.float32)
for _h in range(4):
    _EXPA_np[_h, 32 * _h:32 * _h + 32] = 1.0
    _EXPB_np[4 + _h, 32 * _h:32 * _h + 32] = 1.0
_EXPA = jnp.asarray(_EXPA_np)
_EXPB = jnp.asarray(_EXPB_np)
_SUM1 = jnp.asarray(_SUM1_np)
_SUM2 = jnp.asarray(_SUM2_np)
_S1all = jnp.asarray(_S1all_np)
_SCAT1 = jnp.asarray(_SCAT1_np)
_S2all = jnp.asarray(_S2all_np)
_SCAT2 = jnp.asarray(_SCAT2_np)


_BN = 1000


def _dense_body(x_ref, rbf_ref, rsh_ref, env_ref,
                wr1, wr2, ws1, ws2, wq, bq, wk, bk, wv1, bv1, wv2, bv2,
                exp48, exp8t,
                q_ref, k_ref, v0_ref, v1_ref, m0_ref, m1_ref,
                mq_ref, mk_ref):
    @pl.when(pl.program_id(0) == 0)
    def _init():
        mq_ref[...] = jnp.zeros_like(mq_ref)
        mk_ref[...] = jnp.zeros_like(mk_ref)
    x = x_ref[...]
    t = jnp.dot(rbf_ref[...], wr1[...], preferred_element_type=_F32)
    xsrc = x * env_ref[...] * jnp.dot(t, wr2[...], preferred_element_type=_F32)
    qv = jnp.dot(xsrc, wq[...], preferred_element_type=_F32) + bq[...]
    kv = jnp.dot(x, wk[...], preferred_element_type=_F32) + bk[...]
    q_ref[...] = qv
    k_ref[...] = kv
    qn = jnp.dot(qv * qv, exp8t[...], preferred_element_type=_F32)
    kn = jnp.dot(kv * kv, exp8t[...], preferred_element_type=_F32)
    mq_ref[...] = jnp.maximum(mq_ref[...], jnp.max(qn, axis=0, keepdims=True))
    mk_ref[...] = jnp.maximum(mk_ref[...], jnp.max(kn, axis=0, keepdims=True))
    h = jnp.dot(xsrc, wv1[...], preferred_element_type=_F32) + bv1[...]
    h = h * jax.nn.sigmoid(h)
    vv = jnp.dot(h, wv2[...], preferred_element_type=_F32) + bv2[...]
    v0_ref[...] = vv[:, :128]
    v1_ref[...] = vv[:, 128:]
    scal = jnp.dot(jnp.dot(x, ws1[...], preferred_element_type=_F32),
                   ws2[...], preferred_element_type=_F32)
    mixed = rsh_ref[...] * jnp.dot(scal, exp48[...],
                                   preferred_element_type=_F32)
    m0_ref[:, :96] = mixed[:, :96]
    m0_ref[:, 96:128] = jnp.zeros_like(mixed[:, :32])
    m1_ref[:, :80] = mixed[:, 96:176]
    m1_ref[:, 80:128] = jnp.zeros_like(mixed[:, :48])


def _blk(shape):
    return pl.BlockSpec(shape, lambda i: (i, 0))


def _full(shape):
    return pl.BlockSpec(shape, lambda i: (0, 0))


def _dense_stage(x_scalar, rbf, rsh, envelop_para,
                 W_rbf1, W_rbf2, W_s1, W_s2, W_q, b_q, W_k, b_k,
                 W_v1, b_v1, W_v2, b_v2):
    grid = (N_NODES // _BN,)
    in_specs = [
        _blk((_BN, C)), _blk((_BN, RBF_DIM)), _blk((_BN, IRR_DIM)),
        _blk((_BN, 1)),
        _full((RBF_DIM, C)), _full((C, C)), _full((C, C)),
        _full((C, NUM_IR)), _full((C, C)), _full((1, C)),
        _full((C, C)), _full((1, C)), _full((C, C)), _full((1, C)),
        _full((C, C)), _full((1, C)), _full((NUM_IR, IRR_DIM)),
        _full((C, HEADS)),
    ]
    out_specs = [_blk((_BN, C)), _blk((_BN, C)),
                 _blk((_BN, 128)), _blk((_BN, 128)),
                 _blk((_BN, 128)), _blk((_BN, 128)),
                 _full((1, HEADS)), _full((1, HEADS))]
    out_shape = ([jax.ShapeDtypeStruct((N_NODES, C), _F32)] * 2
                 + [jax.ShapeDtypeStruct((N_NODES, 128), _F32)] * 4
                 + [jax.ShapeDtypeStruct((1, HEADS), _F32)] * 2)
    return pl.pallas_call(
        _dense_body, grid=grid, in_specs=in_specs, out_specs=out_specs,
        out_shape=out_shape,
    )(x_scalar, rbf, rsh, envelop_para,
      W_rbf1, W_rbf2, W_s1, W_s2, W_q, b_q.reshape(1, C), W_k,
      b_k.reshape(1, C), W_v1, b_v1.reshape(1, C), W_v2, b_v2.reshape(1, C),
      _EXP48, _EXP8T)


_CH = 128
_NCH = -(-N_EDGES // (16 * _CH))
_EPW = _NCH * _CH
_EP = 16 * _EPW
_RPW = 632
_NPAD = 16 * _RPW
_DUMMY = N_NODES

_SC_MESH = plsc.VectorSubcoreMesh(core_axis_name="c", subcore_axis_name="s")


def _agg_body(m2_hbm, src3, dst3, zrows, out_hbm, sidx, didx, rows, acc, sem):
    c = lax.axis_index("c")
    s = lax.axis_index("s")
    pltpu.sync_copy(zrows, acc.at[pl.ds(s * _RPW, _RPW)])
    plsc.subcore_barrier()

    def chunk(j, carry):
        pltpu.sync_copy(src3.at[s, j], sidx)
        pltpu.sync_copy(dst3.at[s, j], didx)
        pltpu.async_copy(m2_hbm.at[c].at[sidx], rows, sem).wait()
        pltpu.sync_copy(rows, acc.at[didx], add=True)
        return carry

    lax.fori_loop(0, _NCH, chunk, 0)
    plsc.subcore_barrier()
    pltpu.sync_copy(acc.at[pl.ds(s * _RPW, _RPW)],
                    out_hbm.at[c, pl.ds(s * _RPW, _RPW)])


def _agg_stage(m2, src3, dst3, zrows):
    return pl.kernel(
        _agg_body,
        out_type=jax.ShapeDtypeStruct((2, _NPAD, 128), _F32),
        mesh=_SC_MESH,
        scratch_types=[
            pltpu.VMEM((_CH,), jnp.int32),
            pltpu.VMEM((_CH,), jnp.int32),
            pltpu.VMEM((_CH, 128), _F32),
            pltpu.VMEM_SHARED((_NPAD, 128), _F32),
            pltpu.SemaphoreType.DMA,
        ],
    )(m2, src3, dst3, zrows)


def _den_body(dst3, e128, zrows, outd, didx, ebuf128, accd, sem):
    c = lax.axis_index("c")
    s = lax.axis_index("s")
    pltpu.sync_copy(zrows, accd.at[pl.ds(s * _RPW, _RPW)])
    plsc.subcore_barrier()

    def chunk(j, carry):
        pltpu.sync_copy(dst3.at[s, j], didx)
        pltpu.sync_copy(e128.at[c, pl.ds(s * _EPW + j * _CH, _CH)], ebuf128)
        pltpu.sync_copy(ebuf128, accd.at[didx], add=True)
        return carry

    lax.fori_loop(0, _NCH, chunk, 0)
    plsc.subcore_barrier()
    pltpu.sync_copy(accd.at[pl.ds(s * _RPW, _RPW)],
                    outd.at[c, pl.ds(s * _RPW, _RPW)])


def _den_stage(dst3, e128, zrows):
    return pl.kernel(
        _den_body,
        out_type=jax.ShapeDtypeStruct((2, _NPAD, 128), _F32),
        mesh=_SC_MESH,
        scratch_types=[
            pltpu.VMEM((_CH,), jnp.int32),
            pltpu.VMEM((_CH, 128), _F32),
            pltpu.VMEM_SHARED((_NPAD, 128), _F32),
            pltpu.SemaphoreType.DMA,
        ],
    )(dst3, e128, zrows)


def _qk_body(qk_hbm, gidx, out_hbm, sidx, rows, sem):
    c = lax.axis_index("c")
    s = lax.axis_index("s")

    def chunk(j, carry):
        pltpu.sync_copy(gidx.at[c, s, j], sidx)
        pltpu.async_copy(qk_hbm.at[c].at[sidx], rows, sem).wait()
        pltpu.sync_copy(rows, out_hbm.at[c, pl.ds(s * _EPW + j * _CH, _CH)])
        return carry

    lax.fori_loop(0, _NCH, chunk, 0)


def _qk_stage(qk, gidx):
    return pl.kernel(
        _qk_body,
        out_type=jax.ShapeDtypeStruct((2, _EP, C), _F32),
        mesh=_SC_MESH,
        scratch_types=[
            pltpu.VMEM((_CH,), jnp.int32),
            pltpu.VMEM((_CH, C), _F32),
            pltpu.SemaphoreType.DMA,
        ],
    )(qk, gidx)


_EB = 1024
_NEB = _EP // _EB


def _att_body(qd_ref, ks_ref, mq_ref, mk_ref, exp8t, expa, expb,
              e16_ref, ea_ref, eb_ref):
    p = qd_ref[0] * ks_ref[0]
    att = jnp.dot(p, exp8t[...], preferred_element_type=_F32) * (
        1.0 / math.sqrt(MATCH))
    shift = jnp.sqrt(mq_ref[...] * mk_ref[...]) * (1.0 / math.sqrt(MATCH))
    e = jnp.exp(att - shift)
    e16_ref[:, :HEADS] = e
    e16_ref[:, HEADS:] = jnp.zeros_like(e)
    ea_ref[...] = jnp.dot(e, expa[...], preferred_element_type=_F32)
    eb_ref[...] = jnp.dot(e, expb[...], preferred_element_type=_F32)


def _att_stage(qdks, mq, mk):
    grid = (_NEB,)
    in_specs = [
        pl.BlockSpec((1, _EB, C), lambda i: (0, i, 0)),
        pl.BlockSpec((1, _EB, C), lambda i: (1, i, 0)),
        pl.BlockSpec((1, HEADS), lambda i: (0, 0)),
        pl.BlockSpec((1, HEADS), lambda i: (0, 0)),
        pl.BlockSpec((C, HEADS), lambda i: (0, 0)),
        pl.BlockSpec((HEADS, 128), lambda i: (0, 0)),
        pl.BlockSpec((HEADS, 128), lambda i: (0, 0)),
    ]
    out_specs = [pl.BlockSpec((_EB, 16), lambda i: (i, 0)),
                 pl.BlockSpec((_EB, 128), lambda i: (i, 0)),
                 pl.BlockSpec((_EB, 128), lambda i: (i, 0))]
    return pl.pallas_call(
        _att_body, grid=grid, in_specs=in_specs, out_specs=out_specs,
        out_shape=[jax.ShapeDtypeStruct((_EP, 16), _F32),
                   jax.ShapeDtypeStruct((_EP, 128), _F32),
                   jax.ShapeDtypeStruct((_EP, 128), _F32)],
    )(qdks, qdks, mq, mk, _EXP8T, _EXPA, _EXPB)


def _val_body(v2, src3, dst3, e128, zrv, outv,
              sidx, didx, rows, ebuf128, accv, sem):
    c = lax.axis_index("c")
    s = lax.axis_index("s")
    pltpu.sync_copy(zrv, accv.at[pl.ds(s * _RPW, _RPW)])
    plsc.subcore_barrier()

    def chunk(j, carry):
        pltpu.sync_copy(src3.at[s, j], sidx)
        pltpu.sync_copy(dst3.at[s, j], didx)
        pltpu.async_copy(v2.at[c].at[sidx], rows, sem).wait()
        base = s * _EPW + j * _CH
        pltpu.sync_copy(e128.at[c, pl.ds(base, _CH)], ebuf128)

        def edge(i, ecarry):
            for seg in range(8):
                sl = pl.ds(16 * seg, 16)
                rows[i, sl] = rows[i, sl] * ebuf128[i, sl]
            return ecarry

        lax.fori_loop(0, _CH, edge, 0)
        pltpu.sync_copy(rows, accv.at[didx], add=True)
        return carry

    lax.fori_loop(0, _NCH, chunk, 0)
    plsc.subcore_barrier()
    pltpu.sync_copy(accv.at[pl.ds(s * _RPW, _RPW)],
                    outv.at[c, pl.ds(s * _RPW, _RPW)])


def _val_stage(v2, src3, dst3, e128, zrv):
    return pl.kernel(
        _val_body,
        out_type=jax.ShapeDtypeStruct((2, _NPAD, 128), _F32),
        mesh=_SC_MESH,
        scratch_types=[
            pltpu.VMEM((_CH,), jnp.int32),
            pltpu.VMEM((_CH,), jnp.int32),
            pltpu.VMEM((_CH, 128), _F32),
            pltpu.VMEM((_CH, 128), _F32),
            pltpu.VMEM_SHARED((_NPAD, 128), _F32),
            pltpu.SemaphoreType.DMA,
        ],
    )(v2, src3, dst3, e128, zrv)


def _final_body(sv0_ref, sv1_ref, d0_ref, d1_ref, a0_ref, a1_ref,
                x_ref, xv_ref,
                sum1, sum2, s1all, scat1, s2all, scat2,
                so_ref, vo_ref):
    x = x_ref[...]
    so_ref[:, :128] = sv0_ref[0] / (d0_ref[0] + 1e-16) + x[:, :128]
    so_ref[:, 128:] = sv1_ref[0] / (d1_ref[0] + 1e-16) + x[:, 128:]

    xv = xv_ref[...]
    agg1 = a0_ref[0][:, :96]
    agg2 = a1_ref[0][:, :80]
    b1 = xv[:, :96]
    b2 = xv[:, 96:176]
    A1 = jnp.dot(agg1, sum1[...], preferred_element_type=_F32)
    A2 = jnp.dot(agg2, sum2[...], preferred_element_type=_F32)
    B1 = jnp.dot(b1, s1all[...], preferred_element_type=_F32)
    B2 = jnp.dot(b2, s2all[...], preferred_element_type=_F32)

    cols1 = [jnp.zeros((x.shape[0], 32), _F32) for _ in range(3)]
    for (i, j, k, w) in _T1:
        cols1[k] = cols1[k] + (_C1 * w) * A1[:, i:i + 1] * B1[:, 32 * j:32 * j + 32]
    o1 = jnp.dot(jnp.concatenate(cols1, axis=1), scat1[...],
                 preferred_element_type=_F32)

    cols2 = [jnp.zeros((x.shape[0], 16), _F32) for _ in range(5)]
    for (i, j, k, w) in _T2:
        cols2[k] = cols2[k] + (_C2 * w) * A2[:, i:i + 1] * B2[:, 16 * j:16 * j + 16]
    o2 = jnp.dot(jnp.concatenate(cols2, axis=1), scat2[...],
                 preferred_element_type=_F32)

    vo_ref[:, :96] = o1 + xv[:, :96]
    vo_ref[:, 96:176] = o2 + xv[:, 96:176]


def _final_stage(outv, outd, agg_out, x_scalar, x_vector):
    grid = (N_NODES // _BN,)
    in_specs = [
        pl.BlockSpec((1, _BN, 128), lambda i: (0, i, 0)),
        pl.BlockSpec((1, _BN, 128), lambda i: (1, i, 0)),
        pl.BlockSpec((1, _BN, 128), lambda i: (0, i, 0)),
        pl.BlockSpec((1, _BN, 128), lambda i: (1, i, 0)),
        pl.BlockSpec((1, _BN, 128), lambda i: (0, i, 0)),
        pl.BlockSpec((1, _BN, 128), lambda i: (1, i, 0)),
        _blk((_BN, C)), _blk((_BN, IRR_DIM)),
        _full((96, 8)), _full((80, 8)),
        _full((96, 96)), _full((96, 96)), _full((80, 80)), _full((80, 80)),
    ]
    out_specs = [_blk((_BN, C)), _blk((_BN, IRR_DIM))]
    out_shape = [jax.ShapeDtypeStruct((N_NODES, C), _F32),
                 jax.ShapeDtypeStruct((N_NODES, IRR_DIM), _F32)]
    return pl.pallas_call(
        _final_body, grid=grid, in_specs=in_specs, out_specs=out_specs,
        out_shape=out_shape,
    )(outv, outv, outd, outd, agg_out, agg_out, x_scalar, x_vector,
      _SUM1, _SUM2, _S1all, _SCAT1, _S2all, _SCAT2)


def kernel(x_scalar, x_vector, edge_index, rbf, rsh, envelop_para,
           W_rbf1, W_rbf2, W_s1, W_s2, W_q, b_q, W_k, b_k,
           W_v1, b_v1, W_v2, b_v2):
    src = edge_index[0]
    dst = edge_index[1]

    q, k, v0, v1, m0, m1, mq, mk = _dense_stage(
        x_scalar, rbf, rsh, envelop_para,
        W_rbf1, W_rbf2, W_s1, W_s2, W_q, b_q, W_k, b_k,
        W_v1, b_v1, W_v2, b_v2)

    pad = _EP - N_EDGES
    zpad = jnp.zeros((pad,), jnp.int32)
    src3 = jnp.concatenate([src, zpad]).reshape(16, _NCH, _CH)
    dst3 = jnp.concatenate(
        [dst, jnp.full((pad,), _DUMMY, jnp.int32)]).reshape(16, _NCH, _CH)
    gdst3 = jnp.concatenate([dst, zpad]).reshape(16, _NCH, _CH)
    gidx = jnp.stack([gdst3, src3])

    qk = jnp.stack([q, k])
    qdks = _qk_stage(qk, gidx)
    e16, ea, eb = _att_stage(qdks, mq, mk)

    v2 = jnp.stack([v0, v1])
    zrv = jnp.zeros((_RPW, 128), _F32)
    zrd = jnp.zeros((_RPW, 16), _F32)
    e128 = jnp.stack([ea, eb])
    outv = _val_stage(v2, src3, dst3, e128, zrv)
    outd = _den_stage(dst3, e128, zrv)

    m2 = jnp.stack([m0, m1])
    agg_out = _agg_stage(m2, src3, dst3, zrv)

    return _final_stage(outv, outd, agg_out, x_scalar, x_vector)
